# baseline (speedup 1.0000x reference)
import functools

import jax
import jax.numpy as jnp
from jax import lax
from jax.experimental import pallas as pl
from jax.experimental.pallas import tpu as pltpu
from jax.experimental.pallas import tpu_sc as plsc

VOCAB_SIZE = 1_000_000
EMBED_DIM = 64
BATCH = 4096
SEQ_LEN = 200
MAX_WAVELENGTH = 10000.0

NUM_CORES = 2
NUM_SUBCORES = 16
NW = NUM_CORES * NUM_SUBCORES
ROWS_PER_W = BATCH // NW
CHUNK = SEQ_LEN // 2
NCHUNK = ROWS_PER_W * 2
LANES = 16


def _pos_encoding():
    position = jnp.arange(SEQ_LEN, dtype=jnp.float32)
    min_freq = 1.0 / MAX_WAVELENGTH
    timescales = jnp.power(
        min_freq,
        (2.0 * (jnp.arange(EMBED_DIM, dtype=jnp.float32) // 2)) / float(EMBED_DIM),
    )
    angles = position[:, None] * timescales[None, :]
    cos_mask = jnp.asarray(jnp.arange(EMBED_DIM) % 2, dtype=jnp.float32)
    sin_mask = 1.0 - cos_mask
    return jnp.sin(angles) * sin_mask + jnp.cos(angles) * cos_mask


_mesh = plsc.VectorSubcoreMesh(core_axis_name="c", subcore_axis_name="s")


@functools.partial(
    pl.kernel,
    out_type=jax.ShapeDtypeStruct((NW * NCHUNK, CHUNK, EMBED_DIM), jnp.float32),
    mesh=_mesh,
    compiler_params=pltpu.CompilerParams(use_tc_tiling_on_sc=False),
    scratch_types=[
        pltpu.VMEM((NCHUNK, CHUNK), jnp.int32),
        pltpu.VMEM((SEQ_LEN, EMBED_DIM), jnp.float32),
        pltpu.VMEM((CHUNK, EMBED_DIM), jnp.float32),
        pltpu.SemaphoreType.DMA,
    ],
)
def _emb_kernel(x_hbm, table_hbm, pos_hbm, out_hbm, idx_v, pos_v, rows_v, sem):
    wid = lax.axis_index("s") * NUM_CORES + lax.axis_index("c")
    pltpu.sync_copy(x_hbm.at[wid], idx_v)
    pltpu.sync_copy(pos_hbm, pos_v)

    def chunk_body(j, carry):
        pltpu.async_copy(table_hbm.at[idx_v.at[j]], rows_v, sem).wait()
        prow = (j & 1) * CHUNK

        def row_body(r, c):
            for q in range(EMBED_DIM // LANES):
                sl = pl.ds(q * LANES, LANES)
                rows_v[r, sl] = rows_v[r, sl] + pos_v[prow + r, sl]
            return c

        lax.fori_loop(0, CHUNK, row_body, 0, unroll=2)
        pltpu.sync_copy(rows_v, out_hbm.at[wid * NCHUNK + j])
        return carry

    lax.fori_loop(0, NCHUNK, chunk_body, 0)


def kernel(x, token_emb_table):
    pos = _pos_encoding()
    x_r = x.astype(jnp.int32).reshape(NW, NCHUNK, CHUNK)
    out = _emb_kernel(x_r, token_emb_table, pos)
    return out.reshape(BATCH, SEQ_LEN, EMBED_DIM)


# speedup vs baseline: 1.2044x; 1.2044x over previous
import functools

import jax
import jax.numpy as jnp
from jax import lax
from jax.experimental import pallas as pl
from jax.experimental.pallas import tpu as pltpu
from jax.experimental.pallas import tpu_sc as plsc

VOCAB_SIZE = 1_000_000
EMBED_DIM = 64
BATCH = 4096
SEQ_LEN = 200
MAX_WAVELENGTH = 10000.0

NUM_CORES = 2
NUM_SUBCORES = 16
NW = NUM_CORES * NUM_SUBCORES
RPW = BATCH // NW
HALF = SEQ_LEN // 2
NBUF = 4
LANES = 16


def _pos_encoding():
    position = jnp.arange(SEQ_LEN, dtype=jnp.float32)
    min_freq = 1.0 / MAX_WAVELENGTH
    timescales = jnp.power(
        min_freq,
        (2.0 * (jnp.arange(EMBED_DIM, dtype=jnp.float32) // 2)) / float(EMBED_DIM),
    )
    angles = position[:, None] * timescales[None, :]
    cos_mask = jnp.asarray(jnp.arange(EMBED_DIM) % 2, dtype=jnp.float32)
    sin_mask = 1.0 - cos_mask
    return jnp.sin(angles) * sin_mask + jnp.cos(angles) * cos_mask


_mesh = plsc.VectorSubcoreMesh(core_axis_name="c", subcore_axis_name="s")


@functools.partial(
    pl.kernel,
    out_type=jax.ShapeDtypeStruct((BATCH, SEQ_LEN, EMBED_DIM), jnp.float32),
    mesh=_mesh,
    compiler_params=pltpu.CompilerParams(use_tc_tiling_on_sc=False),
    scratch_types=(
        [pltpu.VMEM((SEQ_LEN, EMBED_DIM), jnp.float32) for _ in range(NBUF)]
        + [
            pltpu.VMEM((2 * RPW, HALF), jnp.int32),
            pltpu.VMEM((SEQ_LEN, EMBED_DIM), jnp.float32),
        ]
        + [pltpu.SemaphoreType.DMA] * (2 * NBUF)
        + [pltpu.SemaphoreType.DMA] * NBUF
    ),
)
def _emb_kernel(x_hbm, table_hbm, pos_hbm, out_hbm, *scratch):
    rows = scratch[:NBUF]
    idx_v = scratch[NBUF]
    pos_v = scratch[NBUF + 1]
    sg = scratch[NBUF + 2 : NBUF + 2 + 2 * NBUF]
    so = scratch[NBUF + 2 + 2 * NBUF :]

    wid = lax.axis_index("s") * NUM_CORES + lax.axis_index("c")
    base = wid * RPW
    pltpu.sync_copy(x_hbm.at[wid], idx_v)
    pltpu.sync_copy(pos_hbm, pos_v)

    def fire_gathers(rb, a):
        for h in range(2):
            pltpu.async_copy(
                table_hbm.at[idx_v.at[2 * rb + h]],
                rows[a].at[pl.ds(h * HALF, HALF)],
                sg[2 * a + h],
            )

    def wait_gathers(rb, a):
        for h in range(2):
            pltpu.make_async_copy(
                table_hbm.at[idx_v.at[2 * rb + h]],
                rows[a].at[pl.ds(h * HALF, HALF)],
                sg[2 * a + h],
            ).wait()

    def wait_out(a):
        pltpu.make_async_copy(rows[a], out_hbm.at[base], so[a]).wait()

    fire_gathers(0, 0)
    fire_gathers(1, 1)

    def outer(t, carry):
        for a in range(NBUF):
            rb = t * NBUF + a
            wait_gathers(rb, a)

            def add_row(r, c):
                for q in range(EMBED_DIM // LANES):
                    sl = pl.ds(q * LANES, LANES)
                    rows[a][r, sl] = rows[a][r, sl] + pos_v[r, sl]
                return c

            lax.fori_loop(0, SEQ_LEN, add_row, 0, unroll=4)
            pltpu.async_copy(rows[a], out_hbm.at[base + rb], so[a])

            rn = rb + 2
            an = (a + 2) % NBUF

            @pl.when(rn < RPW)
            def _():
                @pl.when(rn >= NBUF)
                def _():
                    wait_out(an)

                fire_gathers(rn, an)

        return carry

    lax.fori_loop(0, RPW // NBUF, outer, 0)

    for a in range(NBUF):
        wait_out(a)


def kernel(x, token_emb_table):
    pos = _pos_encoding()
    x_r = x.astype(jnp.int32).reshape(NW, 2 * RPW, HALF)
    return _emb_kernel(x_r, token_emb_table, pos)
